# baseline (speedup 1.0000x reference)
import jax
import jax.numpy as jnp
from jax.experimental import pallas as pl
from jax.experimental.pallas import tpu as pltpu

_NSPLIT = 5
_TS = 200


def _knnconv_body(*refs):
    x_refs = refs[:_NSPLIT]
    w_ref, b_ref, o_ref = refs[_NSPLIT:]
    w = w_ref[...]
    b = b_ref[...]
    for c, x_ref in enumerate(x_refs):
        ts, k, d = x_ref.shape
        x = x_ref[...].reshape(ts * k, d)
        h = jax.lax.dot_general(
            x, w,
            dimension_numbers=(((1,), (1,)), ((), ())),
            preferred_element_type=jnp.float32,
        )
        h = h.reshape(ts, k, h.shape[-1])
        pooled = jnp.max(h, axis=1) + b
        o_ref[c * ts:(c + 1) * ts, :] = jnp.maximum(pooled, 0.0)


def kernel(agg_feat, W0, b0):
    n, k, d = agg_feat.shape
    o = W0.shape[0]
    tn = _NSPLIT * _TS
    grid = n // tn
    b2 = b0.reshape(1, o)
    in_specs = [
        pl.BlockSpec((_TS, k, d), lambda i, c=c: (_NSPLIT * i + c, 0, 0))
        for c in range(_NSPLIT)
    ]
    in_specs += [
        pl.BlockSpec((o, d), lambda i: (0, 0)),
        pl.BlockSpec((1, o), lambda i: (0, 0)),
    ]
    return pl.pallas_call(
        _knnconv_body,
        grid=(grid,),
        in_specs=in_specs,
        out_specs=pl.BlockSpec((tn, o), lambda i: (i, 0)),
        out_shape=jax.ShapeDtypeStruct((n, o), jnp.float32),
        compiler_params=pltpu.CompilerParams(vmem_limit_bytes=128 * 1024 * 1024),
    )(*([agg_feat] * _NSPLIT), W0, b2)
